# Initial kernel scaffold; baseline (speedup 1.0000x reference)
import jax
import jax.numpy as jnp
from jax.experimental import pallas as pl
from jax.experimental.pallas import tpu as pltpu

D = 128
H = 6
NA = 384
NL = 640
B = 8
NA_PER = 48
NL_PER = 80
N = NA + NL
NPS = NA_PER + NL_PER

PH = jax.lax.Precision.HIGHEST


def _dot(a, b):
    return jax.lax.dot(a, b, precision=PH)


def _encoder_body(nodes_ref, ids_ref, Wq_ref, bq_ref, Wk_ref, bk_ref,
                  Wv_ref, bv_ref, Wo1_ref, bo1_ref, Wo2_ref, W1_ref,
                  gamma_ref, beta_ref, W2_ref,
                  a_out_ref, l_out_ref,
                  logits_scr):
    f32 = jnp.float32
    nodes = nodes_ref[:]
    scale = D ** (-0.5)

    Q = _dot(nodes, Wq_ref[:]) + bq_ref[:]
    K = _dot(nodes, Wk_ref[:]) + bk_ref[:]
    V = jax.nn.relu(_dot(nodes, Wv_ref[:]) + bv_ref[:])

    iota_row = jax.lax.broadcasted_iota(jnp.int32, (NPS, N), 1)
    iota_col = jax.lax.broadcasted_iota(jnp.int32, (N, NPS), 0)

    M = jnp.float32(-jnp.inf)
    for b in range(B):
        ids_b = ids_ref[b, :]
        P = (ids_b[:, None] == iota_row).astype(f32)
        Qb = _dot(P, Q)
        Kb = _dot(P, K)
        for h in range(H):
            sl = slice(h * D, (h + 1) * D)
            lg = jax.lax.dot_general(
                Qb[:, sl], Kb[:, sl],
                (((1,), (1,)), ((), ())), precision=PH) * scale
            logits_scr[b, h] = lg
            M = jnp.maximum(M, jnp.max(lg))

    att_sum = jnp.zeros((N, 8), f32)
    O = jnp.zeros((N, H * D), f32)
    for b in range(B):
        ids_b = ids_ref[b, :]
        P = (ids_b[:, None] == iota_row).astype(f32)
        Pt = (iota_col == ids_b[None, :]).astype(f32)
        Vb = _dot(P, V)
        s_cols = []
        o_cols = []
        for h in range(H):
            sl = slice(h * D, (h + 1) * D)
            att = jnp.exp(logits_scr[b, h] - M)
            s_cols.append(jnp.sum(att, axis=1)[:, None])
            o_cols.append(_dot(att, Vb[:, sl]))
        s_cols.append(jnp.zeros((NPS, 8 - H), f32))
        Sb = jnp.concatenate(s_cols, axis=1)
        Ob = jnp.concatenate(o_cols, axis=1)
        att_sum = att_sum + _dot(Pt, Sb)
        O = O + _dot(Pt, Ob)

    denom = jnp.maximum(att_sum, jnp.float32(1e-30))
    o_cols = [O[:, h * D:(h + 1) * D] / denom[:, h:h + 1] for h in range(H)]
    O = jnp.concatenate(o_cols, axis=1)

    out = _dot(jax.nn.relu(_dot(O, Wo1_ref[:]) + bo1_ref[:]), Wo2_ref[:])
    x = _dot(nodes, W1_ref[:]) + out
    mu = jnp.mean(x, axis=-1, keepdims=True)
    var = jnp.mean((x - mu) * (x - mu), axis=-1, keepdims=True)
    x = (x - mu) * jax.lax.rsqrt(var + 1e-5) * gamma_ref[:] + beta_ref[:]
    x = jax.nn.relu(x)
    x = _dot(x, W2_ref[:])
    x = jax.nn.relu(x + nodes)

    iota_a = jax.lax.broadcasted_iota(jnp.int32, (NA_PER, N), 1)
    iota_l = jax.lax.broadcasted_iota(jnp.int32, (NL_PER, N), 1)
    for b in range(B):
        ida = ids_ref[b, 0:NA_PER]
        Pa = (ida[:, None] == iota_a).astype(f32)
        a_out_ref[b * NA_PER:(b + 1) * NA_PER, :] = _dot(Pa, x)
        idl = ids_ref[b, NA_PER:NPS]
        Pl = (idl[:, None] == iota_l).astype(f32)
        l_out_ref[b * NL_PER:(b + 1) * NL_PER, :] = _dot(Pl, x)


@jax.jit
def kernel(agents, agent_ids, lanes, lane_ids, Wq, bq, Wk, bk, Wv, bv,
           Wo1, bo1, Wo2, W1, gamma, beta, W2):
    nodes = jnp.concatenate([agents, lanes], axis=0)
    ids_all = jnp.concatenate([agent_ids, lane_ids + NA], axis=1)

    out = pl.pallas_call(
        _encoder_body,
        out_shape=[
            jax.ShapeDtypeStruct((B * NA_PER, D), jnp.float32),
            jax.ShapeDtypeStruct((B * NL_PER, D), jnp.float32),
        ],
        scratch_shapes=[pltpu.VMEM((B, H, NPS, NPS), jnp.float32)],
    )(nodes, ids_all.astype(jnp.int32),
      Wq, bq.reshape(1, -1), Wk, bk.reshape(1, -1), Wv, bv.reshape(1, -1),
      Wo1, bo1.reshape(1, -1), Wo2, W1,
      gamma.reshape(1, -1), beta.reshape(1, -1), W2)
    return (out[0], out[1])


# speedup vs baseline: 481.3231x; 481.3231x over previous
import jax
import jax.numpy as jnp
from jax.experimental import pallas as pl
from jax.experimental.pallas import tpu as pltpu

D = 128
H = 6
NA = 384
NL = 640
B = 8
NA_PER = 48
NL_PER = 80
N = NA + NL
NPS = NA_PER + NL_PER

PH = jax.lax.Precision.DEFAULT


def _dot(a, b):
    return jax.lax.dot(a, b, precision=PH)


def _encoder_body(nodes_ref, ids_ref, Wq_ref, bq_ref, Wk_ref, bk_ref,
                  Wv_ref, bv_ref, Wo1_ref, bo1_ref, Wo2_ref, W1_ref,
                  gamma_ref, beta_ref, W2_ref,
                  a_out_ref, l_out_ref,
                  logits_scr):
    f32 = jnp.float32
    nodes = nodes_ref[:]
    scale = D ** (-0.5)

    Q = _dot(nodes, Wq_ref[:]) + bq_ref[:]
    K = _dot(nodes, Wk_ref[:]) + bk_ref[:]
    V = jax.nn.relu(_dot(nodes, Wv_ref[:]) + bv_ref[:])

    iota_row = jax.lax.broadcasted_iota(jnp.int32, (NPS, N), 1)
    iota_col = jax.lax.broadcasted_iota(jnp.int32, (N, NPS), 0)

    M = jnp.float32(-jnp.inf)
    for b in range(B):
        ids_b = ids_ref[b, :]
        P = (ids_b[:, None] == iota_row).astype(f32)
        Qb = _dot(P, Q)
        Kb = _dot(P, K)
        for h in range(H):
            sl = slice(h * D, (h + 1) * D)
            lg = jax.lax.dot_general(
                Qb[:, sl], Kb[:, sl],
                (((1,), (1,)), ((), ())), precision=PH) * scale
            logits_scr[b, h] = lg
            M = jnp.maximum(M, jnp.max(lg))

    att_sum = jnp.zeros((N, 128), f32)
    O = jnp.zeros((N, H * D), f32)
    for b in range(B):
        ids_b = ids_ref[b, :]
        P = (ids_b[:, None] == iota_row).astype(f32)
        Pt = (iota_col == ids_b[None, :]).astype(f32)
        Vb = _dot(P, V)
        s_cols = []
        o_cols = []
        for h in range(H):
            sl = slice(h * D, (h + 1) * D)
            att = jnp.exp(logits_scr[b, h] - M)
            s_cols.append(jnp.sum(att, axis=1)[:, None])
            o_cols.append(_dot(att, Vb[:, sl]))
        s_cols.append(jnp.zeros((NPS, 128 - H), f32))
        Sb = jnp.concatenate(s_cols, axis=1)
        Ob = jnp.concatenate(o_cols, axis=1)
        att_sum = att_sum + _dot(Pt, Sb)
        O = O + _dot(Pt, Ob)

    denom = jnp.maximum(att_sum, jnp.float32(1e-30))
    o_cols = [O[:, h * D:(h + 1) * D] / denom[:, h:h + 1] for h in range(H)]
    O = jnp.concatenate(o_cols, axis=1)

    out = _dot(jax.nn.relu(_dot(O, Wo1_ref[:]) + bo1_ref[:]), Wo2_ref[:])
    x = _dot(nodes, W1_ref[:]) + out
    mu = jnp.mean(x, axis=-1, keepdims=True)
    var = jnp.mean((x - mu) * (x - mu), axis=-1, keepdims=True)
    x = (x - mu) * jax.lax.rsqrt(var + 1e-5) * gamma_ref[:] + beta_ref[:]
    x = jax.nn.relu(x)
    x = _dot(x, W2_ref[:])

    ii = jax.lax.broadcasted_iota(jnp.int32, (D, D), 0)
    jj = jax.lax.broadcasted_iota(jnp.int32, (D, D), 1)
    eye = (ii == jj).astype(f32)
    y = _dot(jnp.concatenate([x, nodes], axis=1),
             jnp.concatenate([eye, eye], axis=0))

    iota_a = jax.lax.broadcasted_iota(jnp.int32, (NA_PER, N), 1)
    iota_l = jax.lax.broadcasted_iota(jnp.int32, (NL_PER, N), 1)
    for b in range(B):
        ida = ids_ref[b, 0:NA_PER]
        Pa = (ida[:, None] == iota_a).astype(f32)
        a_out_ref[b * NA_PER:(b + 1) * NA_PER, :] = jnp.maximum(_dot(Pa, y), 0.0)
        idl = ids_ref[b, NA_PER:NPS]
        Pl = (idl[:, None] == iota_l).astype(f32)
        l_out_ref[b * NL_PER:(b + 1) * NL_PER, :] = jnp.maximum(_dot(Pl, y), 0.0)


@jax.jit
def kernel(agents, agent_ids, lanes, lane_ids, Wq, bq, Wk, bk, Wv, bv,
           Wo1, bo1, Wo2, W1, gamma, beta, W2):
    nodes = jnp.concatenate([agents, lanes], axis=0)
    ids_all = jnp.concatenate([agent_ids, lane_ids + NA], axis=1)

    out = pl.pallas_call(
        _encoder_body,
        out_shape=[
            jax.ShapeDtypeStruct((B * NA_PER, D), jnp.float32),
            jax.ShapeDtypeStruct((B * NL_PER, D), jnp.float32),
        ],
        scratch_shapes=[pltpu.VMEM((B, H, NPS, NPS), jnp.float32)],
    )(nodes, ids_all.astype(jnp.int32),
      Wq, bq.reshape(1, -1), Wk, bk.reshape(1, -1), Wv, bv.reshape(1, -1),
      Wo1, bo1.reshape(1, -1), Wo2, W1,
      gamma.reshape(1, -1), beta.reshape(1, -1), W2)
    return (out[0], out[1])


# speedup vs baseline: 616.0515x; 1.2799x over previous
import jax
import jax.numpy as jnp
from jax.experimental import pallas as pl
from jax.experimental.pallas import tpu as pltpu

D = 128
H = 6
NA = 384
NL = 640
B = 8
NA_PER = 48
NL_PER = 80
N = NA + NL
NPS = NA_PER + NL_PER
E = B * NPS

PH = jax.lax.Precision.DEFAULT


def _dot(a, b):
    return jax.lax.dot(a, b, precision=PH)


def _encoder_body(nodes_ref, ids_ref, Wqkv_ref, bqkv_ref,
                  Wo1_ref, bo1_ref, Wo2_ref, W1_ref,
                  gamma_ref, beta_ref, W2_ref,
                  a_out_ref, l_out_ref,
                  logits_scr):
    f32 = jnp.float32
    nodes = nodes_ref[:]
    scale = D ** (-0.5)

    iota_r = jax.lax.broadcasted_iota(jnp.int32, (NPS, N), 1)
    iota_c = jax.lax.broadcasted_iota(jnp.int32, (N, NPS), 0)
    P_all = jnp.concatenate(
        [(ids_ref[b, :][:, None] == iota_r).astype(f32) for b in range(B)],
        axis=0)
    Pt_all = jnp.concatenate(
        [(iota_c == ids_ref[b, :][None, :]).astype(f32) for b in range(B)],
        axis=1)

    G = _dot(P_all, nodes)
    QKV = _dot(G, Wqkv_ref[:]) + bqkv_ref[:]
    Qa = QKV[:, 0:H * D]
    Ka = QKV[:, H * D:2 * H * D]
    Va = jax.nn.relu(QKV[:, 2 * H * D:3 * H * D])

    M = jnp.float32(-jnp.inf)
    for b in range(B):
        rows = slice(b * NPS, (b + 1) * NPS)
        for h in range(H):
            sl = slice(h * D, (h + 1) * D)
            lg = jax.lax.dot_general(
                Qa[rows, sl], Ka[rows, sl],
                (((1,), (1,)), ((), ())), precision=PH) * scale
            logits_scr[b, h] = lg
            M = jnp.maximum(M, jnp.max(lg))

    so_rows = []
    for b in range(B):
        rows = slice(b * NPS, (b + 1) * NPS)
        s_cols = []
        o_cols = []
        for h in range(H):
            sl = slice(h * D, (h + 1) * D)
            att = jnp.exp(logits_scr[b, h] - M)
            s_cols.append(jnp.sum(att, axis=1)[:, None])
            o_cols.append(_dot(att, Va[rows, sl]))
        s_cols.append(jnp.zeros((NPS, D - H), f32))
        so_rows.append(jnp.concatenate(s_cols + o_cols, axis=1))
    SO = jnp.concatenate(so_rows, axis=0)

    R = _dot(Pt_all, SO)
    att_sum = R[:, 0:D]
    denom = jnp.maximum(att_sum, jnp.float32(1e-30))
    o_cols = [R[:, D + h * D:D + (h + 1) * D] / denom[:, h:h + 1]
              for h in range(H)]
    O = jnp.concatenate(o_cols, axis=1)

    out = _dot(jax.nn.relu(_dot(O, Wo1_ref[:]) + bo1_ref[:]), Wo2_ref[:])
    x = _dot(nodes, W1_ref[:]) + out
    mu = jnp.mean(x, axis=-1, keepdims=True)
    var = jnp.mean((x - mu) * (x - mu), axis=-1, keepdims=True)
    x = (x - mu) * jax.lax.rsqrt(var + 1e-5) * gamma_ref[:] + beta_ref[:]
    x = jax.nn.relu(x)
    x = _dot(x, W2_ref[:])

    ii = jax.lax.broadcasted_iota(jnp.int32, (D, D), 0)
    jj = jax.lax.broadcasted_iota(jnp.int32, (D, D), 1)
    eye = (ii == jj).astype(f32)
    y = _dot(jnp.concatenate([x, nodes], axis=1),
             jnp.concatenate([eye, eye], axis=0))

    ysel = jnp.maximum(_dot(P_all, y), 0.0)
    for b in range(B):
        a_out_ref[b * NA_PER:(b + 1) * NA_PER, :] = \
            ysel[b * NPS:b * NPS + NA_PER, :]
        l_out_ref[b * NL_PER:(b + 1) * NL_PER, :] = \
            ysel[b * NPS + NA_PER:(b + 1) * NPS, :]


@jax.jit
def kernel(agents, agent_ids, lanes, lane_ids, Wq, bq, Wk, bk, Wv, bv,
           Wo1, bo1, Wo2, W1, gamma, beta, W2):
    nodes = jnp.concatenate([agents, lanes], axis=0)
    ids_all = jnp.concatenate([agent_ids, lane_ids + NA], axis=1)
    Wqkv = jnp.concatenate([Wq, Wk, Wv], axis=1)
    bqkv = jnp.concatenate([bq, bk, bv]).reshape(1, -1)

    out = pl.pallas_call(
        _encoder_body,
        out_shape=[
            jax.ShapeDtypeStruct((B * NA_PER, D), jnp.float32),
            jax.ShapeDtypeStruct((B * NL_PER, D), jnp.float32),
        ],
        scratch_shapes=[pltpu.VMEM((B, H, NPS, NPS), jnp.float32)],
    )(nodes, ids_all.astype(jnp.int32), Wqkv, bqkv,
      Wo1, bo1.reshape(1, -1), Wo2, W1,
      gamma.reshape(1, -1), beta.reshape(1, -1), W2)
    return (out[0], out[1])
